# baseline (speedup 1.0000x reference)
import functools

import jax
import jax.numpy as jnp
import numpy as np
from jax import lax
from jax.experimental import pallas as pl
from jax.experimental.pallas import tpu as pltpu
from jax.experimental.pallas import tpu_sc as plsc

_NC = 2
_NS = 16


def kernel(inputs, dist_embedding):
    batch, seq = inputs.shape[0], inputs.shape[1]
    emb = dist_embedding.shape[1]
    total = seq * (seq + 1) // 2
    nrows = batch * total
    nw = _NC * _NS
    per_w = nrows // nw
    assert per_w * nw == nrows and per_w % 8 == 0

    chunk = 64
    nfull = per_w // chunk
    tail = per_w - nfull * chunk
    npair = nfull // 2

    idx_np = np.concatenate(
        [np.arange(n, dtype=np.int32) for n in range(seq, 0, -1)])
    idx_all = jnp.asarray(np.tile(idx_np, batch))

    mesh = plsc.VectorSubcoreMesh(core_axis_name="c", subcore_axis_name="s")

    @functools.partial(
        pl.kernel,
        mesh=mesh,
        out_type=jax.ShapeDtypeStruct((nrows, emb), jnp.float32),
        scratch_types=[
            pltpu.VMEM_SHARED((seq, emb), jnp.float32),
            pltpu.VMEM((per_w,), jnp.int32),
            pltpu.VMEM((chunk, emb), jnp.float32),
            pltpu.VMEM((chunk, emb), jnp.float32),
            pltpu.SemaphoreType.DMA,
            pltpu.SemaphoreType.DMA,
            pltpu.SemaphoreType.DMA,
            pltpu.SemaphoreType.DMA,
        ],
        compiler_params=pltpu.CompilerParams(use_tc_tiling_on_sc=False),
    )
    def _gather_kernel(table_hbm, idx_hbm, out_hbm, spmem, idx_v,
                       buf0, buf1, g0, g1, s0, s1):
        sid = lax.axis_index("s")
        wid = lax.axis_index("c") * _NS + sid
        base = wid * per_w

        rows_per = seq // _NS
        pltpu.sync_copy(
            table_hbm.at[pl.ds(sid * rows_per, rows_per)],
            spmem.at[pl.ds(sid * rows_per, rows_per)],
        )
        pltpu.sync_copy(idx_hbm.at[pl.ds(base, per_w)], idx_v)
        plsc.subcore_barrier()

        def gather(c, buf, sem):
            return pltpu.async_copy(
                spmem.at[idx_v.at[pl.ds(c * chunk, chunk)]], buf, sem)

        def scatter(c, buf, sem):
            return pltpu.async_copy(
                buf, out_hbm.at[pl.ds(base + c * chunk, chunk)], sem)

        def wait_chunk(buf, sem):
            pltpu.make_async_copy(table_hbm.at[pl.ds(0, chunk)], buf, sem).wait()

        gather(0, buf0, g0)
        gather(1, buf1, g1)

        def body(j, carry):
            c0 = 2 * j
            wait_chunk(buf0, g0)
            scatter(c0, buf0, s0)
            wait_chunk(buf1, g1)
            scatter(c0 + 1, buf1, s1)

            @pl.when(j < npair - 1)
            def _():
                wait_chunk(buf0, s0)
                gather(c0 + 2, buf0, g0)
                wait_chunk(buf1, s1)
                gather(c0 + 3, buf1, g1)

            return carry

        lax.fori_loop(0, npair, body, 0)

        wait_chunk(buf0, s0)
        wait_chunk(buf1, s1)
        toff = nfull * chunk
        pltpu.async_copy(
            spmem.at[idx_v.at[pl.ds(toff, tail)]],
            buf0.at[pl.ds(0, tail)], g0).wait()
        pltpu.async_copy(
            buf0.at[pl.ds(0, tail)],
            out_hbm.at[pl.ds(base + toff, tail)], s0).wait()

    out = _gather_kernel(dist_embedding, idx_all)
    return out.reshape(batch, total, emb)
